# baseline (speedup 1.0000x reference)
import functools

import jax
import jax.numpy as jnp
from jax import lax
from jax.experimental import pallas as pl
from jax.experimental.pallas import tpu as pltpu
from jax.experimental.pallas import tpu_sc as plsc

N = 10000
E = 320000
D = 128

NUM_CORES = 2
NUM_SUBCORES = 16
NUM_WORKERS = NUM_CORES * NUM_SUBCORES
CHUNK = 128
CHUNKS_PER_TILE = 80
HALF_CHUNKS = 40
E_PAD = NUM_WORKERS * CHUNKS_PER_TILE * CHUNK
N_PAD = 10240
ROWS_PER_TILE = N_PAD // NUM_SUBCORES


def _sc_agg_body(h_hbm, src_hbm, dst_hbm, zeros_hbm, out_hbm,
                 src_v, dst_v, rows_v0, agg_sh):
    c = lax.axis_index("c")
    s = lax.axis_index("s")
    wid = s * NUM_CORES + c

    pltpu.sync_copy(zeros_hbm.at[pl.ds(s * ROWS_PER_TILE, ROWS_PER_TILE)],
                    agg_sh.at[pl.ds(s * ROWS_PER_TILE, ROWS_PER_TILE)])
    pltpu.sync_copy(src_hbm.at[wid], src_v)
    pltpu.sync_copy(dst_hbm.at[wid], dst_v)
    plsc.subcore_barrier()

    def body(j, carry):
        pltpu.sync_copy(h_hbm.at[src_v.at[j]], rows_v0)
        pltpu.sync_copy(rows_v0, agg_sh.at[dst_v.at[j]], add=True)
        return carry

    lax.fori_loop(0, CHUNKS_PER_TILE, body, 0, unroll=False)
    plsc.subcore_barrier()

    pltpu.sync_copy(agg_sh.at[pl.ds(s * ROWS_PER_TILE, ROWS_PER_TILE)],
                    out_hbm.at[c, pl.ds(s * ROWS_PER_TILE, ROWS_PER_TILE)])


_sc_agg = pl.kernel(
    _sc_agg_body,
    out_type=jax.ShapeDtypeStruct((NUM_CORES, N_PAD, D), jnp.float32),
    mesh=plsc.VectorSubcoreMesh(core_axis_name="c", subcore_axis_name="s"),
    scratch_types=[
        pltpu.VMEM((CHUNKS_PER_TILE, CHUNK), jnp.int32),
        pltpu.VMEM((CHUNKS_PER_TILE, CHUNK), jnp.int32),
        pltpu.VMEM((CHUNK, D), jnp.float32),
        pltpu.VMEM_SHARED((N_PAD, D), jnp.float32),
    ],
)


def _init_mm_kernel(x_ref, w_ref, o_ref):
    o_ref[...] = jnp.dot(x_ref[...], w_ref[...],
                         preferred_element_type=jnp.float32)


def _layer_kernel(a0_ref, a1_ref, h_ref, w_ref, b_ref, wr_ref, br_ref, o_ref):
    agg = a0_ref[...] + a1_ref[...]
    conv = jnp.maximum(
        jnp.dot(agg, w_ref[...], preferred_element_type=jnp.float32)
        + b_ref[...], 0.0)
    res = jnp.maximum(
        jnp.dot(h_ref[...], wr_ref[...], preferred_element_type=jnp.float32)
        + br_ref[...], 0.0)
    o_ref[...] = conv + res


_ROW_BLK = 1000
_GRID = N // _ROW_BLK

_init_mm = pl.pallas_call(
    _init_mm_kernel,
    grid=(_GRID,),
    in_specs=[
        pl.BlockSpec((_ROW_BLK, D), lambda i: (i, 0)),
        pl.BlockSpec((D, D), lambda i: (0, 0)),
    ],
    out_specs=pl.BlockSpec((_ROW_BLK, D), lambda i: (i, 0)),
    out_shape=jax.ShapeDtypeStruct((N, D), jnp.float32),
)

_layer = pl.pallas_call(
    _layer_kernel,
    grid=(_GRID,),
    in_specs=[
        pl.BlockSpec((_ROW_BLK, D), lambda i: (i, 0)),
        pl.BlockSpec((_ROW_BLK, D), lambda i: (i, 0)),
        pl.BlockSpec((_ROW_BLK, D), lambda i: (i, 0)),
        pl.BlockSpec((D, D), lambda i: (0, 0)),
        pl.BlockSpec((1, D), lambda i: (0, 0)),
        pl.BlockSpec((D, D), lambda i: (0, 0)),
        pl.BlockSpec((1, D), lambda i: (0, 0)),
    ],
    out_specs=pl.BlockSpec((_ROW_BLK, D), lambda i: (i, 0)),
    out_shape=jax.ShapeDtypeStruct((N, D), jnp.float32),
)


def kernel(x, edge_index, W_init, W1, b1, Wr1, br1, W2, b2, Wr2, br2):
    pad = E_PAD - E
    src = jnp.concatenate(
        [edge_index[0], jnp.zeros((pad,), jnp.int32)]
    ).reshape(NUM_WORKERS, CHUNKS_PER_TILE, CHUNK)
    dst = jnp.concatenate(
        [edge_index[1], jnp.full((pad,), N_PAD - 1, jnp.int32)]
    ).reshape(NUM_WORKERS, CHUNKS_PER_TILE, CHUNK)
    zeros = jnp.zeros((N_PAD, D), jnp.float32)

    h = _init_mm(x, W_init)
    for (W, b, Wr, br) in ((W1, b1, Wr1, br1), (W2, b2, Wr2, br2)):
        parts = _sc_agg(h, src, dst, zeros)
        h = _layer(parts[0], parts[1], h,
                   W, b.reshape(1, D), Wr, br.reshape(1, D))
    return h.reshape(100, N // 100, D)


# speedup vs baseline: 3.0160x; 3.0160x over previous
import functools

import jax
import jax.numpy as jnp
from jax import lax
from jax.experimental import pallas as pl
from jax.experimental.pallas import tpu as pltpu
from jax.experimental.pallas import tpu_sc as plsc

N = 10000
E = 320000
D = 128

NUM_CORES = 2
NUM_SUBCORES = 16
NUM_WORKERS = NUM_CORES * NUM_SUBCORES
CHUNK = 125
CHUNKS_PER_TILE = 80
HALF_CHUNKS = CHUNKS_PER_TILE // 2
N_PAD = 10240
ROWS_PER_TILE = N_PAD // NUM_SUBCORES


def _sc_agg_body(h_hbm, src_hbm, dst_hbm, zeros_hbm, out_hbm,
                 src_v, dst_v, rows_v0, rows_v1, sem0, sem1, agg_sh):
    c = lax.axis_index("c")
    s = lax.axis_index("s")
    wid = s * NUM_CORES + c

    def g_start(j, buf, sem):
        pltpu.make_async_copy(h_hbm.at[src_v.at[j]], buf, sem).start()

    def g_wait(j, buf, sem):
        pltpu.make_async_copy(h_hbm.at[src_v.at[j]], buf, sem).wait()

    def scat(j, buf):
        pltpu.sync_copy(buf, agg_sh.at[dst_v.at[j]], add=True)

    pltpu.sync_copy(zeros_hbm.at[pl.ds(s * ROWS_PER_TILE, ROWS_PER_TILE)],
                    agg_sh.at[pl.ds(s * ROWS_PER_TILE, ROWS_PER_TILE)])
    plsc.subcore_barrier()

    for half in range(2):
        pltpu.sync_copy(
            src_hbm.at[wid, pl.ds(half * HALF_CHUNKS, HALF_CHUNKS)], src_v)
        pltpu.sync_copy(
            dst_hbm.at[wid, pl.ds(half * HALF_CHUNKS, HALF_CHUNKS)], dst_v)
        g_start(0, rows_v0, sem0)

        def body(i, carry):
            j = 2 * i
            g_wait(j, rows_v0, sem0)
            g_start(j + 1, rows_v1, sem1)
            scat(j, rows_v0)
            g_wait(j + 1, rows_v1, sem1)
            g_start(j + 2, rows_v0, sem0)
            scat(j + 1, rows_v1)
            return carry

        lax.fori_loop(0, HALF_CHUNKS // 2 - 1, body, 0, unroll=False)
        jl = HALF_CHUNKS - 2
        g_wait(jl, rows_v0, sem0)
        g_start(jl + 1, rows_v1, sem1)
        scat(jl, rows_v0)
        g_wait(jl + 1, rows_v1, sem1)
        scat(jl + 1, rows_v1)
    plsc.subcore_barrier()

    pltpu.sync_copy(agg_sh.at[pl.ds(s * ROWS_PER_TILE, ROWS_PER_TILE)],
                    out_hbm.at[c, pl.ds(s * ROWS_PER_TILE, ROWS_PER_TILE)])


_sc_agg = pl.kernel(
    _sc_agg_body,
    out_type=jax.ShapeDtypeStruct((NUM_CORES, N_PAD, D), jnp.float32),
    mesh=plsc.VectorSubcoreMesh(core_axis_name="c", subcore_axis_name="s"),
    scratch_types=[
        pltpu.VMEM((HALF_CHUNKS, CHUNK), jnp.int32),
        pltpu.VMEM((HALF_CHUNKS, CHUNK), jnp.int32),
        pltpu.VMEM((CHUNK, D), jnp.float32),
        pltpu.VMEM((CHUNK, D), jnp.float32),
        pltpu.SemaphoreType.DMA,
        pltpu.SemaphoreType.DMA,
        pltpu.VMEM_SHARED((N_PAD, D), jnp.float32),
    ],
)


def _init_mm_kernel(x_ref, w_ref, o_ref):
    o_ref[...] = jnp.dot(x_ref[...], w_ref[...],
                         preferred_element_type=jnp.float32)


def _layer_kernel(a0_ref, a1_ref, h_ref, w_ref, b_ref, wr_ref, br_ref, o_ref):
    agg = a0_ref[...] + a1_ref[...]
    conv = jnp.maximum(
        jnp.dot(agg, w_ref[...], preferred_element_type=jnp.float32)
        + b_ref[...], 0.0)
    res = jnp.maximum(
        jnp.dot(h_ref[...], wr_ref[...], preferred_element_type=jnp.float32)
        + br_ref[...], 0.0)
    o_ref[...] = conv + res


_ROW_BLK = 1000
_GRID = N // _ROW_BLK

_init_mm = pl.pallas_call(
    _init_mm_kernel,
    grid=(_GRID,),
    in_specs=[
        pl.BlockSpec((_ROW_BLK, D), lambda i: (i, 0)),
        pl.BlockSpec((D, D), lambda i: (0, 0)),
    ],
    out_specs=pl.BlockSpec((_ROW_BLK, D), lambda i: (i, 0)),
    out_shape=jax.ShapeDtypeStruct((N, D), jnp.float32),
)

_layer = pl.pallas_call(
    _layer_kernel,
    grid=(_GRID,),
    in_specs=[
        pl.BlockSpec((_ROW_BLK, D), lambda i: (i, 0)),
        pl.BlockSpec((_ROW_BLK, D), lambda i: (i, 0)),
        pl.BlockSpec((_ROW_BLK, D), lambda i: (i, 0)),
        pl.BlockSpec((D, D), lambda i: (0, 0)),
        pl.BlockSpec((1, D), lambda i: (0, 0)),
        pl.BlockSpec((D, D), lambda i: (0, 0)),
        pl.BlockSpec((1, D), lambda i: (0, 0)),
    ],
    out_specs=pl.BlockSpec((_ROW_BLK, D), lambda i: (i, 0)),
    out_shape=jax.ShapeDtypeStruct((N, D), jnp.float32),
)


def kernel(x, edge_index, W_init, W1, b1, Wr1, br1, W2, b2, Wr2, br2):
    src = edge_index[0].reshape(NUM_WORKERS, CHUNKS_PER_TILE, CHUNK)
    dst = edge_index[1].reshape(NUM_WORKERS, CHUNKS_PER_TILE, CHUNK)
    zeros = jnp.zeros((N_PAD, D), jnp.float32)

    h = _init_mm(x, W_init)
    for (W, b, Wr, br) in ((W1, b1, Wr1, br1), (W2, b2, Wr2, br2)):
        parts = _sc_agg(h, src, dst, zeros)
        h = _layer(parts[0], parts[1], h,
                   W, b.reshape(1, D), Wr, br.reshape(1, D))
    return h.reshape(100, N // 100, D)


# speedup vs baseline: 3.1543x; 1.0459x over previous
import functools

import jax
import jax.numpy as jnp
from jax import lax
from jax.experimental import pallas as pl
from jax.experimental.pallas import tpu as pltpu
from jax.experimental.pallas import tpu_sc as plsc

N = 10000
E = 320000
D = 128

NUM_CORES = 2
NUM_SUBCORES = 16
NUM_WORKERS = NUM_CORES * NUM_SUBCORES
CHUNK = 125
CHUNKS_PER_TILE = 80
HALF_CHUNKS = CHUNKS_PER_TILE // 2
N_PAD = 10240
ROWS_PER_TILE = N_PAD // NUM_SUBCORES


def _sc_agg_body(h_hbm, src_hbm, dst_hbm, zeros_hbm, out_hbm,
                 src_v, dst_v, rows_v0, rows_v1, sem0, sem1, agg_sh):
    c = lax.axis_index("c")
    s = lax.axis_index("s")
    wid = s * NUM_CORES + c

    def g_start(j, buf, sem):
        pltpu.make_async_copy(h_hbm.at[src_v.at[j]], buf, sem).start()

    def g_wait(j, buf, sem):
        pltpu.make_async_copy(h_hbm.at[src_v.at[j]], buf, sem).wait()

    def scat(j, buf):
        pltpu.sync_copy(buf, agg_sh.at[dst_v.at[j]], add=True)

    pltpu.sync_copy(zeros_hbm.at[pl.ds(s * ROWS_PER_TILE, ROWS_PER_TILE)],
                    agg_sh.at[pl.ds(s * ROWS_PER_TILE, ROWS_PER_TILE)])
    plsc.subcore_barrier()

    for half in range(2):
        pltpu.sync_copy(
            src_hbm.at[wid, pl.ds(half * HALF_CHUNKS, HALF_CHUNKS)], src_v)
        pltpu.sync_copy(
            dst_hbm.at[wid, pl.ds(half * HALF_CHUNKS, HALF_CHUNKS)], dst_v)
        g_start(0, rows_v0, sem0)

        def body(i, carry):
            j = 2 * i
            g_wait(j, rows_v0, sem0)
            g_start(j + 1, rows_v1, sem1)
            scat(j, rows_v0)
            g_wait(j + 1, rows_v1, sem1)
            g_start(j + 2, rows_v0, sem0)
            scat(j + 1, rows_v1)
            return carry

        lax.fori_loop(0, HALF_CHUNKS // 2 - 1, body, 0, unroll=False)
        jl = HALF_CHUNKS - 2
        g_wait(jl, rows_v0, sem0)
        g_start(jl + 1, rows_v1, sem1)
        scat(jl, rows_v0)
        g_wait(jl + 1, rows_v1, sem1)
        scat(jl + 1, rows_v1)
    plsc.subcore_barrier()

    pltpu.sync_copy(agg_sh.at[pl.ds(s * ROWS_PER_TILE, ROWS_PER_TILE)],
                    out_hbm.at[c, pl.ds(s * ROWS_PER_TILE, ROWS_PER_TILE)])


_sc_agg = pl.kernel(
    _sc_agg_body,
    out_type=jax.ShapeDtypeStruct((NUM_CORES, N_PAD, D), jnp.float32),
    mesh=plsc.VectorSubcoreMesh(core_axis_name="c", subcore_axis_name="s"),
    scratch_types=[
        pltpu.VMEM((HALF_CHUNKS, CHUNK), jnp.int32),
        pltpu.VMEM((HALF_CHUNKS, CHUNK), jnp.int32),
        pltpu.VMEM((CHUNK, D), jnp.float32),
        pltpu.VMEM((CHUNK, D), jnp.float32),
        pltpu.SemaphoreType.DMA,
        pltpu.SemaphoreType.DMA,
        pltpu.VMEM_SHARED((N_PAD, D), jnp.float32),
    ],
)


def _mm_relu_kernel(x_ref, w_ref, b_ref, o_ref):
    o_ref[...] = jnp.maximum(
        jnp.dot(x_ref[...], w_ref[...], preferred_element_type=jnp.float32)
        + b_ref[...], 0.0)


def _init_mm_kernel(x_ref, w_ref, o_ref):
    o_ref[...] = jnp.dot(x_ref[...], w_ref[...],
                         preferred_element_type=jnp.float32)


def _conv_add_kernel(p_ref, w_ref, b_ref, res_ref, o_ref):
    agg = p_ref[0] + p_ref[1]
    conv = jnp.maximum(
        jnp.dot(agg, w_ref[...], preferred_element_type=jnp.float32)
        + b_ref[...], 0.0)
    o_ref[...] = conv + res_ref[...]


_ROW_BLK = 1000
_GRID = N // _ROW_BLK

_init_mm = pl.pallas_call(
    _init_mm_kernel,
    grid=(_GRID,),
    in_specs=[
        pl.BlockSpec((_ROW_BLK, D), lambda i: (i, 0)),
        pl.BlockSpec((D, D), lambda i: (0, 0)),
    ],
    out_specs=pl.BlockSpec((_ROW_BLK, D), lambda i: (i, 0)),
    out_shape=jax.ShapeDtypeStruct((N, D), jnp.float32),
)

_res_mm = pl.pallas_call(
    _mm_relu_kernel,
    grid=(_GRID,),
    in_specs=[
        pl.BlockSpec((_ROW_BLK, D), lambda i: (i, 0)),
        pl.BlockSpec((D, D), lambda i: (0, 0)),
        pl.BlockSpec((1, D), lambda i: (0, 0)),
    ],
    out_specs=pl.BlockSpec((_ROW_BLK, D), lambda i: (i, 0)),
    out_shape=jax.ShapeDtypeStruct((N, D), jnp.float32),
)

_conv_add = pl.pallas_call(
    _conv_add_kernel,
    grid=(_GRID,),
    in_specs=[
        pl.BlockSpec((2, _ROW_BLK, D), lambda i: (0, i, 0)),
        pl.BlockSpec((D, D), lambda i: (0, 0)),
        pl.BlockSpec((1, D), lambda i: (0, 0)),
        pl.BlockSpec((_ROW_BLK, D), lambda i: (i, 0)),
    ],
    out_specs=pl.BlockSpec((_ROW_BLK, D), lambda i: (i, 0)),
    out_shape=jax.ShapeDtypeStruct((N, D), jnp.float32),
)


def kernel(x, edge_index, W_init, W1, b1, Wr1, br1, W2, b2, Wr2, br2):
    src = edge_index[0].reshape(NUM_WORKERS, CHUNKS_PER_TILE, CHUNK)
    dst = edge_index[1].reshape(NUM_WORKERS, CHUNKS_PER_TILE, CHUNK)
    zeros = jnp.zeros((N_PAD, D), jnp.float32)

    h = _init_mm(x, W_init)
    for (W, b, Wr, br) in ((W1, b1, Wr1, br1), (W2, b2, Wr2, br2)):
        parts = _sc_agg(h, src, dst, zeros)
        res = _res_mm(h, Wr, br.reshape(1, D))
        h = _conv_add(parts, W, b.reshape(1, D), res)
    return h.reshape(100, N // 100, D)
